# Initial kernel scaffold; baseline (speedup 1.0000x reference)
import jax
import jax.numpy as jnp
from jax.experimental import pallas as pl


def kernel(x, prototypes):
    raise NotImplementedError("write your pallas kernel here")


# speedup vs baseline: 1.9489x; 1.9489x over previous
import functools

import jax
import jax.numpy as jnp
from jax import lax
from jax.experimental import pallas as pl
from jax.experimental.pallas import tpu as pltpu
from jax.experimental.pallas import tpu_sc as plsc

M_TILE = 512


def _scores_body(x_ref, p_ref, s_ref, idx_ref):
    x = x_ref[...]
    p = p_ref[...]
    cross = lax.dot_general(
        x, p, (((1,), (1,)), ((), ())), preferred_element_type=jnp.float32
    )
    x_sq = jnp.sum(x * x, axis=1, keepdims=True)
    p_sq = jnp.sum(p * p, axis=1)
    dist = x_sq - 2.0 * cross + p_sq[None, :]
    scores = -dist
    s_ref[...] = scores
    idx = jnp.argmax(scores, axis=1).astype(jnp.int32)
    idx_ref[...] = idx.reshape(1, 1, M_TILE)


def _scores_and_argmax(xr, prototypes):
    M, d = xr.shape
    K = prototypes.shape[0]
    n_tiles = M // M_TILE
    scores, idx3 = pl.pallas_call(
        _scores_body,
        grid=(n_tiles,),
        in_specs=[
            pl.BlockSpec((M_TILE, d), lambda i: (i, 0)),
            pl.BlockSpec((K, d), lambda i: (0, 0)),
        ],
        out_specs=[
            pl.BlockSpec((M_TILE, K), lambda i: (i, 0)),
            pl.BlockSpec((1, 1, M_TILE), lambda i: (i, 0, 0)),
        ],
        out_shape=[
            jax.ShapeDtypeStruct((M, K), jnp.float32),
            jax.ShapeDtypeStruct((n_tiles, 1, M_TILE), jnp.int32),
        ],
    )(xr, prototypes)
    return scores, idx3.reshape(M)


def _make_sc_gather(V, D, B):
    info = plsc.get_sparse_core_info()
    NC, NS = info.num_cores, info.num_subcores
    NW = NC * NS
    assert D % info.num_lanes == 0 and B % (8 * NW) == 0
    b_per_w = B // NW
    mesh = plsc.VectorSubcoreMesh(core_axis_name="c", subcore_axis_name="s")

    @functools.partial(
        pl.kernel,
        mesh=mesh,
        out_type=jax.ShapeDtypeStruct((B, D), jnp.float32),
        scratch_types=[
            pltpu.VMEM((b_per_w,), jnp.int32),
            pltpu.VMEM((b_per_w, D), jnp.float32),
            pltpu.SemaphoreType.DMA,
        ],
    )
    def gather(table_hbm, idx_hbm, out_hbm, idx_v, rows_v, sem):
        wid = lax.axis_index("s") * NC + lax.axis_index("c")
        base = wid * b_per_w
        pltpu.sync_copy(idx_hbm.at[pl.ds(base, b_per_w)], idx_v)
        pltpu.async_copy(table_hbm.at[idx_v], rows_v, sem).wait()
        pltpu.sync_copy(rows_v, out_hbm.at[pl.ds(base, b_per_w)])

    return gather


def kernel(x, prototypes):
    B, N, d = x.shape
    K = prototypes.shape[0]
    M = B * N
    xr = x.reshape(M, d)
    scores_flat, idx = _scores_and_argmax(xr, prototypes)
    matched_flat = _make_sc_gather(K, d, M)(prototypes, idx)
    return matched_flat.reshape(B, N, d), scores_flat.reshape(B, N, K)
